# baseline (speedup 1.0000x reference)
import functools

import jax
import jax.numpy as jnp
from jax import lax
from jax.experimental import pallas as pl
from jax.experimental.pallas import tpu as pltpu
from jax.experimental.pallas import tpu_sc as plsc

N_NODES_C = 10000
N_PAD = 10240
N_EDGES_C = 320000
D = 128

NC = 2
NS = 16
NW = NC * NS
CHUNK = 80
N_CHUNKS = 128
E_PER_W = N_CHUNKS * CHUNK
E_PAD = NW * E_PER_W
NBUF = 2
N_GROUPS = N_CHUNKS // NBUF
ROWS_PER_TILE = N_PAD // NS


def _feats_gather():
  mesh = plsc.VectorSubcoreMesh(core_axis_name="c", subcore_axis_name="s")
  RPT = N_PAD // NW

  @functools.partial(
      pl.kernel,
      mesh=mesh,
      compiler_params=pltpu.CompilerParams(needs_layout_passes=False),
      out_type=jax.ShapeDtypeStruct((N_PAD, D), jnp.float32),
      scratch_types=[
          pltpu.VMEM((RPT,), jnp.int32),
          pltpu.VMEM((RPT, D), jnp.float32),
          pltpu.SemaphoreType.DMA,
      ],
  )
  def feats_gather(ids_hbm, table_hbm, out_hbm, idx_v, rows_v, sem):
    wid = lax.axis_index("c") * NS + lax.axis_index("s")
    r0 = wid * RPT
    pltpu.sync_copy(ids_hbm.at[pl.ds(r0, RPT)], idx_v)
    copies = [
        pltpu.make_async_copy(
            table_hbm.at[idx_v.at[pl.ds(t * CHUNK, CHUNK)]],
            rows_v.at[pl.ds(t * CHUNK, CHUNK)], sem)
        for t in range(RPT // CHUNK)
    ]
    for cp in copies:
      cp.start()
    for cp in copies:
      cp.wait()
    pltpu.sync_copy(rows_v, out_hbm.at[pl.ds(r0, RPT)])

  return feats_gather


def _make_edge_layer():
  mesh = plsc.VectorSubcoreMesh(core_axis_name="c", subcore_axis_name="s")

  scratch = [
      pltpu.VMEM((E_PER_W,), jnp.int32),
      pltpu.VMEM((N_CHUNKS, CHUNK), jnp.int32),
      pltpu.VMEM_SHARED((N_PAD, D), jnp.float32)
  ] + [pltpu.VMEM((CHUNK, D), jnp.float32) for _ in range(NBUF)] \
    + [pltpu.SemaphoreType.DMA for _ in range(NBUF)]

  @functools.partial(
      pl.kernel,
      mesh=mesh,
      compiler_params=pltpu.CompilerParams(needs_layout_passes=False),
      out_type=jax.ShapeDtypeStruct((NC, N_PAD, D), jnp.float32),
      scratch_types=scratch,
  )
  def edge_layer(src_hbm, dst_hbm, table_hbm, zeros_hbm, out_hbm,
                 src1d, dst2d, acc, *bufs_and_sems):
    rows = bufs_and_sems[:NBUF]
    sems = bufs_and_sems[NBUF:2 * NBUF]
    c = lax.axis_index("c")
    s = lax.axis_index("s")
    wid = c * NS + s

    row0 = s * ROWS_PER_TILE
    pltpu.sync_copy(zeros_hbm.at[pl.ds(row0, ROWS_PER_TILE)],
                    acc.at[pl.ds(row0, ROWS_PER_TILE)])

    pltpu.sync_copy(src_hbm.at[wid], src1d)
    pltpu.sync_copy(dst_hbm.at[wid], dst2d)
    plsc.subcore_barrier()

    def gather(k, b):
      idx = src1d.at[pl.ds(k * CHUNK, CHUNK)]
      return pltpu.make_async_copy(table_hbm.at[idx], rows[b], sems[b])

    for b in range(NBUF):
      gather(b, b).start()

    def body(g, carry):
      for b in range(NBUF):
        k = g * NBUF + b
        gather(k, b).wait()
        pltpu.sync_copy(rows[b], acc.at[dst2d.at[k]], add=True)

        @pl.when(g < N_GROUPS - 1)
        def _():
          gather(k + NBUF, b).start()
      return carry

    lax.fori_loop(0, N_GROUPS, body, 0)
    plsc.subcore_barrier()

    pltpu.sync_copy(acc.at[pl.ds(row0, ROWS_PER_TILE)],
                    out_hbm.at[c, pl.ds(row0, ROWS_PER_TILE)])

  return edge_layer


_feats = _feats_gather()
_edge_layer = _make_edge_layer()


def _dense_relu_kernel(p_ref, w_ref, b_ref, o_ref):
  h = p_ref[0] + p_ref[1]
  o_ref[...] = jax.nn.relu(
      jnp.dot(h, w_ref[...], preferred_element_type=jnp.float32) + b_ref[...])


def _dense_head_kernel(p_ref, w_ref, b_ref, wo_ref, bo_ref, o_ref):
  h = p_ref[0] + p_ref[1]
  h2 = jax.nn.relu(
      jnp.dot(h, w_ref[...], preferred_element_type=jnp.float32) + b_ref[...])
  o_ref[...] = jax.nn.sigmoid(
      jnp.dot(h2, wo_ref[...], preferred_element_type=jnp.float32)
      + bo_ref[...])


_ROWS_BLK = 2048


def _dense_relu(partials, w, b):
  return pl.pallas_call(
      _dense_relu_kernel,
      grid=(N_PAD // _ROWS_BLK,),
      in_specs=[
          pl.BlockSpec((NC, _ROWS_BLK, D), lambda i: (0, i, 0)),
          pl.BlockSpec((D, D), lambda i: (0, 0)),
          pl.BlockSpec((1, D), lambda i: (0, 0)),
      ],
      out_specs=pl.BlockSpec((_ROWS_BLK, D), lambda i: (i, 0)),
      out_shape=jax.ShapeDtypeStruct((N_PAD, D), jnp.float32),
  )(partials, w, b.reshape(1, D))


def _dense_head(partials, w, b, wout, bout):
  return pl.pallas_call(
      _dense_head_kernel,
      grid=(N_PAD // _ROWS_BLK,),
      in_specs=[
          pl.BlockSpec((NC, _ROWS_BLK, D), lambda i: (0, i, 0)),
          pl.BlockSpec((D, D), lambda i: (0, 0)),
          pl.BlockSpec((1, D), lambda i: (0, 0)),
          pl.BlockSpec((D, 1), lambda i: (0, 0)),
          pl.BlockSpec((1, 1), lambda i: (0, 0)),
      ],
      out_specs=pl.BlockSpec((_ROWS_BLK, 1), lambda i: (i, 0)),
      out_shape=jax.ShapeDtypeStruct((N_PAD, 1), jnp.float32),
  )(partials, w, b.reshape(1, D), wout, bout.reshape(1, 1))


@jax.jit
def kernel(cncpt_ids, edge_index, concept_table, W1, b1, W2, b2, Wout, bout):
  pad = E_PAD - N_EDGES_C
  src = jnp.concatenate(
      [edge_index[0], jnp.zeros((pad,), edge_index.dtype)]
  ).reshape(NW, E_PER_W)
  dst = jnp.concatenate(
      [edge_index[1], jnp.full((pad,), N_NODES_C, edge_index.dtype)]
  ).reshape(NW, N_CHUNKS, CHUNK)
  zeros = jnp.zeros((N_PAD, D), jnp.float32)
  ids = jnp.concatenate(
      [cncpt_ids, jnp.zeros((N_PAD - N_NODES_C,), cncpt_ids.dtype)])

  feats = _feats(ids, concept_table)
  p1 = _edge_layer(src, dst, feats, zeros)
  h1 = _dense_relu(p1, W1, b1)
  p2 = _edge_layer(src, dst, h1, zeros)
  logits = _dense_head(p2, W2, b2, Wout, bout)
  return logits[None, :N_NODES_C, :]


# speedup vs baseline: 1.0001x; 1.0001x over previous
import functools

import jax
import jax.numpy as jnp
from jax import lax
from jax.experimental import pallas as pl
from jax.experimental.pallas import tpu as pltpu
from jax.experimental.pallas import tpu_sc as plsc

N_NODES_C = 10000
N_PAD = 10240
N_EDGES_C = 320000
D = 128

NC = 2
NS = 16
NW = NC * NS
CHUNK = 80
N_CHUNKS = 128
E_PER_W = N_CHUNKS * CHUNK
E_PAD = NW * E_PER_W
NBUF = 2
N_GROUPS = N_CHUNKS // NBUF
ROWS_PER_TILE = N_PAD // NS


def _feats_gather():
  mesh = plsc.VectorSubcoreMesh(core_axis_name="c", subcore_axis_name="s")
  RPT = N_PAD // NW

  @functools.partial(
      pl.kernel,
      mesh=mesh,
      compiler_params=pltpu.CompilerParams(needs_layout_passes=False),
      out_type=jax.ShapeDtypeStruct((N_PAD, D), jnp.float32),
      scratch_types=[
          pltpu.VMEM((RPT,), jnp.int32),
          pltpu.VMEM((RPT, D), jnp.float32),
          pltpu.SemaphoreType.DMA,
      ],
  )
  def feats_gather(ids_hbm, table_hbm, out_hbm, idx_v, rows_v, sem):
    wid = lax.axis_index("c") * NS + lax.axis_index("s")
    r0 = wid * RPT
    pltpu.sync_copy(ids_hbm.at[pl.ds(r0, RPT)], idx_v)
    copies = [
        pltpu.make_async_copy(
            table_hbm.at[idx_v.at[pl.ds(t * CHUNK, CHUNK)]],
            rows_v.at[pl.ds(t * CHUNK, CHUNK)], sem)
        for t in range(RPT // CHUNK)
    ]
    for cp in copies:
      cp.start()
    for cp in copies:
      cp.wait()
    pltpu.sync_copy(rows_v, out_hbm.at[pl.ds(r0, RPT)])

  return feats_gather


def _make_edge_layer():
  mesh = plsc.VectorSubcoreMesh(core_axis_name="c", subcore_axis_name="s")

  scratch = [
      pltpu.VMEM((E_PER_W,), jnp.int32),
      pltpu.VMEM((N_CHUNKS, CHUNK), jnp.int32),
      pltpu.VMEM_SHARED((N_PAD, D), jnp.float32)
  ] + [pltpu.VMEM((CHUNK, D), jnp.float32) for _ in range(NBUF)] \
    + [pltpu.SemaphoreType.DMA for _ in range(NBUF)]

  @functools.partial(
      pl.kernel,
      mesh=mesh,
      compiler_params=pltpu.CompilerParams(needs_layout_passes=False),
      out_type=jax.ShapeDtypeStruct((NC, N_PAD, D), jnp.float32),
      scratch_types=scratch,
  )
  def edge_layer(src_hbm, dst_hbm, table_hbm, zeros_hbm, out_hbm,
                 src1d, dst2d, acc, *bufs_and_sems):
    rows = bufs_and_sems[:NBUF]
    sems = bufs_and_sems[NBUF:2 * NBUF]
    c = lax.axis_index("c")
    s = lax.axis_index("s")
    wid = c * NS + s

    row0 = s * ROWS_PER_TILE
    pltpu.sync_copy(zeros_hbm.at[pl.ds(row0, ROWS_PER_TILE)],
                    acc.at[pl.ds(row0, ROWS_PER_TILE)])

    pltpu.sync_copy(src_hbm.at[wid], src1d)
    pltpu.sync_copy(dst_hbm.at[wid], dst2d)
    plsc.subcore_barrier()

    def gather(k, b):
      idx = src1d.at[pl.ds(k * CHUNK, CHUNK)]
      return pltpu.make_async_copy(table_hbm.at[idx], rows[b], sems[b])

    for b in range(NBUF):
      gather(b, b).start()

    def body(g, carry):
      for b in range(NBUF):
        k = g * NBUF + b
        gather(k, b).wait()
        pltpu.sync_copy(rows[b], acc.at[dst2d.at[k]], add=True)

        @pl.when(g < N_GROUPS - 1)
        def _():
          gather(k + NBUF, b).start()
      return carry

    lax.fori_loop(0, N_GROUPS, body, 0)
    plsc.subcore_barrier()

    pltpu.sync_copy(acc.at[pl.ds(row0, ROWS_PER_TILE)],
                    out_hbm.at[c, pl.ds(row0, ROWS_PER_TILE)])

  return edge_layer


_feats = _feats_gather()
_edge_layer = _make_edge_layer()


def _dense_relu_kernel(p_ref, w_ref, b_ref, o_ref):
  h = p_ref[0] + p_ref[1]
  o_ref[...] = jax.nn.relu(
      jnp.dot(h, w_ref[...], preferred_element_type=jnp.float32) + b_ref[...])


def _dense_head_kernel(p_ref, w_ref, b_ref, wo_ref, bo_ref, o_ref):
  h = p_ref[0] + p_ref[1]
  h2 = jax.nn.relu(
      jnp.dot(h, w_ref[...], preferred_element_type=jnp.float32) + b_ref[...])
  o_ref[...] = jax.nn.sigmoid(
      jnp.dot(h2, wo_ref[...], preferred_element_type=jnp.float32)
      + bo_ref[...])


_ROWS_BLK = 2048


def _dense_relu(partials, w, b):
  return pl.pallas_call(
      _dense_relu_kernel,
      grid=(N_PAD // _ROWS_BLK,),
      in_specs=[
          pl.BlockSpec((NC, _ROWS_BLK, D), lambda i: (0, i, 0)),
          pl.BlockSpec((D, D), lambda i: (0, 0)),
          pl.BlockSpec((1, D), lambda i: (0, 0)),
      ],
      out_specs=pl.BlockSpec((_ROWS_BLK, D), lambda i: (i, 0)),
      out_shape=jax.ShapeDtypeStruct((N_PAD, D), jnp.float32),
  )(partials, w, b.reshape(1, D))


def _dense_head(partials, w, b, wout, bout):
  return pl.pallas_call(
      _dense_head_kernel,
      grid=(N_PAD // _ROWS_BLK,),
      in_specs=[
          pl.BlockSpec((NC, _ROWS_BLK, D), lambda i: (0, i, 0)),
          pl.BlockSpec((D, D), lambda i: (0, 0)),
          pl.BlockSpec((1, D), lambda i: (0, 0)),
          pl.BlockSpec((D, 1), lambda i: (0, 0)),
          pl.BlockSpec((1, 1), lambda i: (0, 0)),
      ],
      out_specs=pl.BlockSpec((_ROWS_BLK, 1), lambda i: (i, 0)),
      out_shape=jax.ShapeDtypeStruct((N_PAD, 1), jnp.float32),
  )(partials, w, b.reshape(1, D), wout, bout.reshape(1, 1))


@jax.jit
def kernel(cncpt_ids, edge_index, concept_table, W1, b1, W2, b2, Wout, bout):
  pad = E_PAD - N_EDGES_C
  src = jnp.concatenate(
      [edge_index[0], jnp.zeros((pad,), edge_index.dtype)]
  ).reshape(NW, E_PER_W)
  pad_dst = N_NODES_C + jnp.arange(pad, dtype=edge_index.dtype) % (
      N_PAD - N_NODES_C)
  dst = jnp.concatenate(
      [edge_index[1], pad_dst]).reshape(NW, N_CHUNKS, CHUNK)
  zeros = jnp.zeros((N_PAD, D), jnp.float32)
  ids = jnp.concatenate(
      [cncpt_ids, jnp.zeros((N_PAD - N_NODES_C,), cncpt_ids.dtype)])

  feats = _feats(ids, concept_table)
  p1 = _edge_layer(src, dst, feats, zeros)
  h1 = _dense_relu(p1, W1, b1)
  p2 = _edge_layer(src, dst, h1, zeros)
  logits = _dense_head(p2, W2, b2, Wout, bout)
  return logits[None, :N_NODES_C, :]


# speedup vs baseline: 1.2156x; 1.2155x over previous
import functools

import jax
import jax.numpy as jnp
from jax import lax
from jax.experimental import pallas as pl
from jax.experimental.pallas import tpu as pltpu
from jax.experimental.pallas import tpu_sc as plsc

N_NODES_C = 10000
N_PAD = 10240
N_EDGES_C = 320000
D = 128

NC = 2
NS = 16
CHUNK = 80
N_CHUNKS_TOT = 4096
E_PAD = N_CHUNKS_TOT * CHUNK
K0 = 200
K1 = 56
NBUF = 2
ROWS_PER_TILE = N_PAD // NS
E_ALLOC = E_PAD + K0 * CHUNK


def _make_edge_layer(compose: bool):
  mesh = plsc.VectorSubcoreMesh(core_axis_name="c", subcore_axis_name="s")

  scratch = [
      pltpu.VMEM((K0 * CHUNK,), jnp.int32),
      pltpu.VMEM_SHARED((N_PAD, D), jnp.float32)
  ] + ([pltpu.VMEM((N_NODES_C,), jnp.int32)] if compose else []) \
    + [pltpu.VMEM((CHUNK, D), jnp.float32) for _ in range(NBUF)] \
    + [pltpu.VMEM((CHUNK,), jnp.int32) for _ in range(NBUF)] \
    + [pltpu.SemaphoreType.DMA for _ in range(NBUF)] \
    + [pltpu.SemaphoreType.DMA for _ in range(NBUF)]

  @functools.partial(
      pl.kernel,
      mesh=mesh,
      compiler_params=pltpu.CompilerParams(needs_layout_passes=False),
      out_type=jax.ShapeDtypeStruct((NC, N_PAD, D), jnp.float32),
      scratch_types=scratch,
  )
  def edge_layer(ids_hbm, src_hbm, dst_hbm, table_hbm, zeros_hbm, out_hbm,
                 src1d, acc, *rest):
    if compose:
      ids_v, rest = rest[0], rest[1:]
    rows = rest[:NBUF]
    dstb = rest[NBUF:2 * NBUF]
    semg = rest[2 * NBUF:3 * NBUF]
    semi = rest[3 * NBUF:4 * NBUF]
    c = lax.axis_index("c")
    s = lax.axis_index("s")
    n_chunks = jnp.where(c == 0, K0, K1)
    chunk0 = jnp.where(c == 0, s * K0, NS * K0 + s * K1)
    e0 = chunk0 * CHUNK

    row0 = s * ROWS_PER_TILE
    pltpu.sync_copy(zeros_hbm.at[pl.ds(row0, ROWS_PER_TILE)],
                    acc.at[pl.ds(row0, ROWS_PER_TILE)])
    pltpu.sync_copy(src_hbm.at[pl.ds(e0, K0 * CHUNK)], src1d)
    if compose:
      pltpu.sync_copy(ids_hbm, ids_v)

      def compose_body(k, carry):
        for j in range(CHUNK // 16):
          idx = src1d[pl.ds(k * CHUNK + j * 16, 16)]
          src1d[pl.ds(k * CHUNK + j * 16, 16)] = plsc.load_gather(
              ids_v, [idx])
        return carry

      lax.fori_loop(0, n_chunks, compose_body, 0)
    plsc.subcore_barrier()

    def gather(k, b):
      return pltpu.make_async_copy(
          table_hbm.at[src1d.at[pl.ds(k * CHUNK, CHUNK)]], rows[b], semg[b])

    def dst_copy(k, b):
      return pltpu.make_async_copy(
          dst_hbm.at[pl.ds((chunk0 + k) * CHUNK, CHUNK)], dstb[b], semi[b])

    for b in range(NBUF):
      gather(b, b).start()
      dst_copy(b, b).start()

    def body(g, carry):
      for b in range(NBUF):
        kk = g * NBUF + b
        gather(kk, b).wait()
        dst_copy(kk, b).wait()
        pltpu.sync_copy(rows[b], acc.at[dstb[b]], add=True)

        @pl.when(kk + NBUF < n_chunks)
        def _():
          gather(kk + NBUF, b).start()
          dst_copy(kk + NBUF, b).start()
      return carry

    lax.fori_loop(0, n_chunks // NBUF, body, 0)
    plsc.subcore_barrier()

    pltpu.sync_copy(acc.at[pl.ds(row0, ROWS_PER_TILE)],
                    out_hbm.at[c, pl.ds(row0, ROWS_PER_TILE)])

  return edge_layer


_edge_layer1 = _make_edge_layer(compose=True)
_edge_layer2 = _make_edge_layer(compose=False)


def _dense_relu_kernel(p_ref, w_ref, b_ref, o_ref):
  h = p_ref[0] + p_ref[1]
  o_ref[...] = jax.nn.relu(
      jnp.dot(h, w_ref[...], preferred_element_type=jnp.float32) + b_ref[...])


def _dense_head_kernel(p_ref, w_ref, b_ref, wo_ref, bo_ref, o_ref):
  h = p_ref[0] + p_ref[1]
  h2 = jax.nn.relu(
      jnp.dot(h, w_ref[...], preferred_element_type=jnp.float32) + b_ref[...])
  o_ref[...] = jax.nn.sigmoid(
      jnp.dot(h2, wo_ref[...], preferred_element_type=jnp.float32)
      + bo_ref[...])


_ROWS_BLK = 2048


def _dense_relu(partials, w, b):
  return pl.pallas_call(
      _dense_relu_kernel,
      grid=(N_PAD // _ROWS_BLK,),
      in_specs=[
          pl.BlockSpec((NC, _ROWS_BLK, D), lambda i: (0, i, 0)),
          pl.BlockSpec((D, D), lambda i: (0, 0)),
          pl.BlockSpec((1, D), lambda i: (0, 0)),
      ],
      out_specs=pl.BlockSpec((_ROWS_BLK, D), lambda i: (i, 0)),
      out_shape=jax.ShapeDtypeStruct((N_PAD, D), jnp.float32),
  )(partials, w, b.reshape(1, D))


def _dense_head(partials, w, b, wout, bout):
  return pl.pallas_call(
      _dense_head_kernel,
      grid=(N_PAD // _ROWS_BLK,),
      in_specs=[
          pl.BlockSpec((NC, _ROWS_BLK, D), lambda i: (0, i, 0)),
          pl.BlockSpec((D, D), lambda i: (0, 0)),
          pl.BlockSpec((1, D), lambda i: (0, 0)),
          pl.BlockSpec((D, 1), lambda i: (0, 0)),
          pl.BlockSpec((1, 1), lambda i: (0, 0)),
      ],
      out_specs=pl.BlockSpec((_ROWS_BLK, 1), lambda i: (i, 0)),
      out_shape=jax.ShapeDtypeStruct((N_PAD, 1), jnp.float32),
  )(partials, w, b.reshape(1, D), wout, bout.reshape(1, 1))


@jax.jit
def kernel(cncpt_ids, edge_index, concept_table, W1, b1, W2, b2, Wout, bout):
  pad = E_ALLOC - N_EDGES_C
  src = jnp.concatenate([edge_index[0], jnp.zeros((pad,), edge_index.dtype)])
  pad_dst = N_NODES_C + jnp.arange(pad, dtype=edge_index.dtype) % (
      N_PAD - N_NODES_C)
  dst = jnp.concatenate([edge_index[1], pad_dst])
  zeros = jnp.zeros((N_PAD, D), jnp.float32)

  p1 = _edge_layer1(cncpt_ids, src, dst, concept_table, zeros)
  h1 = _dense_relu(p1, W1, b1)
  p2 = _edge_layer2(cncpt_ids, src, dst, h1, zeros)
  logits = _dense_head(p2, W2, b2, Wout, bout)
  return logits[None, :N_NODES_C, :]


# speedup vs baseline: 1.5440x; 1.2702x over previous
import functools

import jax
import jax.numpy as jnp
from jax import lax
from jax.experimental import pallas as pl
from jax.experimental.pallas import tpu as pltpu
from jax.experimental.pallas import tpu_sc as plsc

N_NODES_C = 10000
N_PAD = 10240
N_EDGES_C = 320000
D = 128

NC = 2
NS = 16
NW = NC * NS
E_PER_W = N_EDGES_C // NW
CHUNK = 80
N_CHUNKS = E_PER_W // CHUNK
ROWS_PER_TILE = N_PAD // NS


def _make_edge_layer(compose: bool):
  mesh = plsc.VectorSubcoreMesh(core_axis_name="c", subcore_axis_name="s")

  scratch = (
      [
          pltpu.VMEM_SHARED((N_PAD, D), jnp.float32),
          pltpu.VMEM((CHUNK,), jnp.int32),
          pltpu.VMEM((CHUNK,), jnp.int32),
          pltpu.VMEM((CHUNK,), jnp.int32),
          pltpu.VMEM((CHUNK, D), jnp.float32),
      ]
      + ([pltpu.VMEM((N_NODES_C,), jnp.int32)] if compose else [])
      + [pltpu.SemaphoreType.DMA]
  )

  @functools.partial(
      pl.kernel,
      mesh=mesh,
      compiler_params=pltpu.CompilerParams(needs_layout_passes=False),
      out_type=jax.ShapeDtypeStruct((NC, N_PAD, D), jnp.float32),
      scratch_types=scratch,
  )
  def edge_layer(ids_hbm, src_hbm, dst_hbm, table_hbm, zeros_hbm, out_hbm,
                 acc, src_v, dst_v, eff_v, rows_v, *rest):
    if compose:
      ids_v, rest = rest[0], rest[1:]
    sem = rest[0]
    c = lax.axis_index("c")
    s = lax.axis_index("s")
    wid = c * NS + s

    row0 = s * ROWS_PER_TILE
    pltpu.sync_copy(zeros_hbm.at[pl.ds(row0, ROWS_PER_TILE)],
                    acc.at[pl.ds(row0, ROWS_PER_TILE)])
    if compose:
      pltpu.sync_copy(ids_hbm, ids_v)
    plsc.subcore_barrier()

    def body(k, carry):
      off = wid * E_PER_W + k * CHUNK
      pltpu.sync_copy(src_hbm.at[pl.ds(off, CHUNK)], src_v)
      pltpu.sync_copy(dst_hbm.at[pl.ds(off, CHUNK)], dst_v)
      if compose:
        for j in range(CHUNK // 16):
          idx = src_v[pl.ds(j * 16, 16)]
          eff_v[pl.ds(j * 16, 16)] = plsc.load_gather(ids_v, [idx])
        gather_idx = eff_v
      else:
        gather_idx = src_v
      pltpu.async_copy(table_hbm.at[gather_idx], rows_v, sem).wait()
      pltpu.sync_copy(rows_v, acc.at[dst_v], add=True)
      return carry

    lax.fori_loop(0, N_CHUNKS, body, 0)
    plsc.subcore_barrier()

    pltpu.sync_copy(acc.at[pl.ds(row0, ROWS_PER_TILE)],
                    out_hbm.at[c, pl.ds(row0, ROWS_PER_TILE)])

  return edge_layer


_edge_layer1 = _make_edge_layer(compose=True)
_edge_layer2 = _make_edge_layer(compose=False)


def _dense_relu_kernel(p_ref, w_ref, b_ref, o_ref):
  h = p_ref[0] + p_ref[1]
  o_ref[...] = jax.nn.relu(
      jnp.dot(h, w_ref[...], preferred_element_type=jnp.float32) + b_ref[...])


def _dense_head_kernel(p_ref, w_ref, b_ref, wo_ref, bo_ref, o_ref):
  h = p_ref[0] + p_ref[1]
  h2 = jax.nn.relu(
      jnp.dot(h, w_ref[...], preferred_element_type=jnp.float32) + b_ref[...])
  o_ref[...] = jax.nn.sigmoid(
      jnp.dot(h2, wo_ref[...], preferred_element_type=jnp.float32)
      + bo_ref[...])


_ROWS_BLK = 2048


def _dense_relu(partials, w, b):
  return pl.pallas_call(
      _dense_relu_kernel,
      grid=(N_PAD // _ROWS_BLK,),
      in_specs=[
          pl.BlockSpec((NC, _ROWS_BLK, D), lambda i: (0, i, 0)),
          pl.BlockSpec((D, D), lambda i: (0, 0)),
          pl.BlockSpec((1, D), lambda i: (0, 0)),
      ],
      out_specs=pl.BlockSpec((_ROWS_BLK, D), lambda i: (i, 0)),
      out_shape=jax.ShapeDtypeStruct((N_PAD, D), jnp.float32),
  )(partials, w, b.reshape(1, D))


def _dense_head(partials, w, b, wout, bout):
  return pl.pallas_call(
      _dense_head_kernel,
      grid=(N_PAD // _ROWS_BLK,),
      in_specs=[
          pl.BlockSpec((NC, _ROWS_BLK, D), lambda i: (0, i, 0)),
          pl.BlockSpec((D, D), lambda i: (0, 0)),
          pl.BlockSpec((1, D), lambda i: (0, 0)),
          pl.BlockSpec((D, 1), lambda i: (0, 0)),
          pl.BlockSpec((1, 1), lambda i: (0, 0)),
      ],
      out_specs=pl.BlockSpec((_ROWS_BLK, 1), lambda i: (i, 0)),
      out_shape=jax.ShapeDtypeStruct((N_PAD, 1), jnp.float32),
  )(partials, w, b.reshape(1, D), wout, bout.reshape(1, 1))


@jax.jit
def kernel(cncpt_ids, edge_index, concept_table, W1, b1, W2, b2, Wout, bout):
  src = edge_index[0]
  dst = edge_index[1]
  zeros = jnp.zeros((N_PAD, D), jnp.float32)

  p1 = _edge_layer1(cncpt_ids, src, dst, concept_table, zeros)
  h1 = _dense_relu(p1, W1, b1)
  p2 = _edge_layer2(cncpt_ids, src, dst, h1, zeros)
  logits = _dense_head(p2, W2, b2, Wout, bout)
  return logits[None, :N_NODES_C, :]


# speedup vs baseline: 1.8217x; 1.1798x over previous
import functools

import jax
import jax.numpy as jnp
from jax import lax
from jax.experimental import pallas as pl
from jax.experimental.pallas import tpu as pltpu
from jax.experimental.pallas import tpu_sc as plsc

N_NODES_C = 10000
N_PAD = 10240
N_EDGES_C = 320000
D = 128

NC = 2
NS = 16
NW = NC * NS
E_PER_W = N_EDGES_C // NW
CHUNK = 80
N_CHUNKS = E_PER_W // CHUNK
ROWS_PER_TILE = N_PAD // NS


def _make_edge_layer(compose: bool):
  mesh = plsc.VectorSubcoreMesh(core_axis_name="c", subcore_axis_name="s")

  scratch = (
      [
          pltpu.VMEM_SHARED((N_PAD, D), jnp.float32),
          pltpu.VMEM((CHUNK,), jnp.int32),
          pltpu.VMEM((CHUNK,), jnp.int32),
          pltpu.VMEM((CHUNK,), jnp.int32),
          pltpu.VMEM((CHUNK, D), jnp.float32),
      ]
      + ([pltpu.VMEM((N_NODES_C,), jnp.int32)] if compose else [])
      + [pltpu.SemaphoreType.DMA, pltpu.SemaphoreType.DMA]
  )

  @functools.partial(
      pl.kernel,
      mesh=mesh,
      compiler_params=pltpu.CompilerParams(needs_layout_passes=False),
      out_type=jax.ShapeDtypeStruct((NC, N_PAD, D), jnp.float32),
      scratch_types=scratch,
  )
  def edge_layer(ids_hbm, src_hbm, dst_hbm, table_hbm, zeros_hbm, out_hbm,
                 acc, src_v, dst_v, eff_v, rows_v, *rest):
    if compose:
      ids_v, rest = rest[0], rest[1:]
    sem, semd = rest[0], rest[1]
    c = lax.axis_index("c")
    s = lax.axis_index("s")
    wid = c * NS + s

    row0 = s * ROWS_PER_TILE
    pltpu.sync_copy(zeros_hbm.at[pl.ds(row0, ROWS_PER_TILE)],
                    acc.at[pl.ds(row0, ROWS_PER_TILE)])
    if compose:
      pltpu.sync_copy(ids_hbm, ids_v)
    plsc.subcore_barrier()

    def body(k, carry):
      off = wid * E_PER_W + k * CHUNK
      dst_cp = pltpu.make_async_copy(dst_hbm.at[pl.ds(off, CHUNK)], dst_v,
                                     semd)
      dst_cp.start()
      pltpu.sync_copy(src_hbm.at[pl.ds(off, CHUNK)], src_v)
      if compose:
        for j in range(CHUNK // 16):
          idx = src_v[pl.ds(j * 16, 16)]
          eff_v[pl.ds(j * 16, 16)] = plsc.load_gather(ids_v, [idx])
        gather_idx = eff_v
      else:
        gather_idx = src_v
      pltpu.async_copy(table_hbm.at[gather_idx], rows_v, sem).wait()
      dst_cp.wait()
      pltpu.sync_copy(rows_v, acc.at[dst_v], add=True)
      return carry

    lax.fori_loop(0, N_CHUNKS, body, 0)
    plsc.subcore_barrier()

    pltpu.sync_copy(acc.at[pl.ds(row0, ROWS_PER_TILE)],
                    out_hbm.at[c, pl.ds(row0, ROWS_PER_TILE)])

  return edge_layer


_edge_layer1 = _make_edge_layer(compose=True)
_edge_layer2 = _make_edge_layer(compose=False)


def _dense_relu_kernel(p_ref, w_ref, b_ref, o_ref):
  h = p_ref[0] + p_ref[1]
  o_ref[...] = jax.nn.relu(
      jnp.dot(h, w_ref[...], preferred_element_type=jnp.float32) + b_ref[...])


def _dense_head_kernel(p_ref, w_ref, b_ref, wo_ref, bo_ref, o_ref):
  h = p_ref[0] + p_ref[1]
  h2 = jax.nn.relu(
      jnp.dot(h, w_ref[...], preferred_element_type=jnp.float32) + b_ref[...])
  o_ref[...] = jax.nn.sigmoid(
      jnp.dot(h2, wo_ref[...], preferred_element_type=jnp.float32)
      + bo_ref[...])


_ROWS_BLK = 2048


def _dense_relu(partials, w, b):
  return pl.pallas_call(
      _dense_relu_kernel,
      grid=(N_PAD // _ROWS_BLK,),
      in_specs=[
          pl.BlockSpec((NC, _ROWS_BLK, D), lambda i: (0, i, 0)),
          pl.BlockSpec((D, D), lambda i: (0, 0)),
          pl.BlockSpec((1, D), lambda i: (0, 0)),
      ],
      out_specs=pl.BlockSpec((_ROWS_BLK, D), lambda i: (i, 0)),
      out_shape=jax.ShapeDtypeStruct((N_PAD, D), jnp.float32),
  )(partials, w, b.reshape(1, D))


def _dense_head(partials, w, b, wout, bout):
  return pl.pallas_call(
      _dense_head_kernel,
      grid=(N_PAD // _ROWS_BLK,),
      in_specs=[
          pl.BlockSpec((NC, _ROWS_BLK, D), lambda i: (0, i, 0)),
          pl.BlockSpec((D, D), lambda i: (0, 0)),
          pl.BlockSpec((1, D), lambda i: (0, 0)),
          pl.BlockSpec((D, 1), lambda i: (0, 0)),
          pl.BlockSpec((1, 1), lambda i: (0, 0)),
      ],
      out_specs=pl.BlockSpec((_ROWS_BLK, 1), lambda i: (i, 0)),
      out_shape=jax.ShapeDtypeStruct((N_PAD, 1), jnp.float32),
  )(partials, w, b.reshape(1, D), wout, bout.reshape(1, 1))


@jax.jit
def kernel(cncpt_ids, edge_index, concept_table, W1, b1, W2, b2, Wout, bout):
  src = edge_index[0]
  dst = edge_index[1]
  zeros = jnp.zeros((N_PAD, D), jnp.float32)

  p1 = _edge_layer1(cncpt_ids, src, dst, concept_table, zeros)
  h1 = _dense_relu(p1, W1, b1)
  p2 = _edge_layer2(cncpt_ids, src, dst, h1, zeros)
  logits = _dense_head(p2, W2, b2, Wout, bout)
  return logits[None, :N_NODES_C, :]
